# Initial kernel scaffold; baseline (speedup 1.0000x reference)
import jax
import jax.numpy as jnp
from jax.experimental import pallas as pl


def kernel(x, weight):
    raise NotImplementedError("write your pallas kernel here")


# speedup vs baseline: 2.2927x; 2.2927x over previous
import functools

import jax
import jax.numpy as jnp
from jax import lax
from jax.experimental import pallas as pl
from jax.experimental.pallas import tpu as pltpu
from jax.experimental.pallas import tpu_sc as plsc

_B = 4
_S = 8192
_D = 768
_N = _B * _S

_NC = 2
_NS = 16
_NW = _NC * _NS
_PER_W = _N // _NW
_CHUNK = 64
_NBUF = 2
_NCHUNK = _PER_W // _CHUNK

_mesh = plsc.VectorSubcoreMesh(core_axis_name="c", subcore_axis_name="s")


@functools.partial(
    pl.kernel,
    mesh=_mesh,
    out_type=jax.ShapeDtypeStruct((_N, _D), jnp.float32),
    scratch_types=[
        pltpu.VMEM((_PER_W,), jnp.int32),
        pltpu.VMEM((_NBUF, _CHUNK, _D), jnp.float32),
        pltpu.SemaphoreType.DMA,
        pltpu.SemaphoreType.DMA,
    ],
)
def _emb_lookup(x_hbm, w_hbm, out_hbm, idx_v, rows_v, sem0, sem1):
    wid = lax.axis_index("s") * _NC + lax.axis_index("c")
    base = wid * _PER_W
    pltpu.sync_copy(x_hbm.at[pl.ds(base, _PER_W)], idx_v)

    sems = (sem0, sem1)
    copies = [None] * _NBUF
    copies[0] = pltpu.async_copy(
        w_hbm.at[idx_v.at[pl.ds(0, _CHUNK)]], rows_v.at[0], sems[0]
    )
    for g in range(_NCHUNK):
        slot = g % _NBUF
        if g + 1 < _NCHUNK:
            nslot = (g + 1) % _NBUF
            copies[nslot] = pltpu.async_copy(
                w_hbm.at[idx_v.at[pl.ds((g + 1) * _CHUNK, _CHUNK)]],
                rows_v.at[nslot],
                sems[nslot],
            )
        copies[slot].wait()
        pltpu.sync_copy(
            rows_v.at[slot], out_hbm.at[pl.ds(base + g * _CHUNK, _CHUNK)]
        )


def kernel(x, weight):
    idx = jnp.reshape(x, (_N,)).astype(jnp.int32)
    out = _emb_lookup(idx, weight)
    return jnp.reshape(out, (_B, _S, _D))


# speedup vs baseline: 2.2937x; 1.0004x over previous
import functools

import jax
import jax.numpy as jnp
from jax import lax
from jax.experimental import pallas as pl
from jax.experimental.pallas import tpu as pltpu
from jax.experimental.pallas import tpu_sc as plsc

_B = 4
_S = 8192
_D = 768
_N = _B * _S

_NC = 2
_NS = 16
_NW = _NC * _NS
_PER_W = _N // _NW
_CHUNK = 32
_NBUF = 4
_NCHUNK = _PER_W // _CHUNK

_mesh = plsc.VectorSubcoreMesh(core_axis_name="c", subcore_axis_name="s")


@functools.partial(
    pl.kernel,
    mesh=_mesh,
    out_type=jax.ShapeDtypeStruct((_N, _D), jnp.float32),
    scratch_types=[
        pltpu.VMEM((_PER_W,), jnp.int32),
        pltpu.VMEM((_NBUF, _CHUNK, _D), jnp.float32),
        [pltpu.SemaphoreType.DMA] * _NBUF,
        [pltpu.SemaphoreType.DMA] * _NBUF,
    ],
)
def _emb_lookup(x_hbm, w_hbm, out_hbm, idx_v, rows_v, gsems, ssems):
    wid = lax.axis_index("s") * _NC + lax.axis_index("c")
    base = wid * _PER_W
    pltpu.sync_copy(x_hbm.at[pl.ds(base, _PER_W)], idx_v)

    def start_gather(g):
        slot = g % _NBUF
        return pltpu.async_copy(
            w_hbm.at[idx_v.at[pl.ds(g * _CHUNK, _CHUNK)]],
            rows_v.at[slot],
            gsems[slot],
        )

    gather = [None] * _NBUF
    store = [None] * _NBUF
    for g in range(_NBUF - 1):
        gather[g % _NBUF] = start_gather(g)
    for g in range(_NCHUNK):
        slot = g % _NBUF
        if g + _NBUF - 1 < _NCHUNK:
            nslot = (g + _NBUF - 1) % _NBUF
            if store[nslot] is not None:
                store[nslot].wait()
            gather[nslot] = start_gather(g + _NBUF - 1)
        gather[slot].wait()
        store[slot] = pltpu.async_copy(
            rows_v.at[slot],
            out_hbm.at[pl.ds(base + g * _CHUNK, _CHUNK)],
            ssems[slot],
        )
    for s in range(_NBUF):
        if store[s] is not None:
            store[s].wait()


def kernel(x, weight):
    idx = jnp.reshape(x, (_N,)).astype(jnp.int32)
    out = _emb_lookup(idx, weight)
    return jnp.reshape(out, (_B, _S, _D))


# speedup vs baseline: 2.2981x; 1.0019x over previous
import functools

import jax
import jax.numpy as jnp
from jax import lax
from jax.experimental import pallas as pl
from jax.experimental.pallas import tpu as pltpu
from jax.experimental.pallas import tpu_sc as plsc

_B = 4
_S = 8192
_D = 768
_N = _B * _S

_NC = 2
_NS = 16
_NW = _NC * _NS
_PER_W = _N // _NW
_CHUNK = 32
_NBUF = 2
_NCHUNK = _PER_W // _CHUNK

_mesh = plsc.VectorSubcoreMesh(core_axis_name="c", subcore_axis_name="s")


@functools.partial(
    pl.kernel,
    mesh=_mesh,
    out_type=jax.ShapeDtypeStruct((_N, _D), jnp.float32),
    scratch_types=[
        pltpu.VMEM((_PER_W,), jnp.int32),
        pltpu.VMEM((_NBUF, _CHUNK, _D), jnp.float32),
        pltpu.VMEM_SHARED((_NS, _NBUF, _CHUNK, _D), jnp.float32),
        [pltpu.SemaphoreType.DMA] * _NBUF,
        [pltpu.SemaphoreType.DMA] * _NBUF,
    ],
)
def _emb_lookup(x_hbm, w_hbm, out_hbm, idx_v, rows_v, rows_sp, gsems, ssems):
    sid = lax.axis_index("s")
    wid = sid * _NC + lax.axis_index("c")
    base = wid * _PER_W
    pltpu.sync_copy(x_hbm.at[pl.ds(base, _PER_W)], idx_v)

    def start_gather(g):
        slot = g % _NBUF
        return pltpu.async_copy(
            w_hbm.at[idx_v.at[pl.ds(g * _CHUNK, _CHUNK)]],
            rows_v.at[slot],
            gsems[slot],
        )

    gather = [None] * _NBUF
    store = [None] * _NBUF
    gather[0] = start_gather(0)
    for g in range(_NCHUNK):
        slot = g % _NBUF
        if g + 1 < _NCHUNK:
            gather[(g + 1) % _NBUF] = start_gather(g + 1)
        gather[slot].wait()
        if store[slot] is not None:
            store[slot].wait()
        pltpu.sync_copy(rows_v.at[slot], rows_sp.at[sid, slot])
        store[slot] = pltpu.async_copy(
            rows_sp.at[sid, slot],
            out_hbm.at[pl.ds(base + g * _CHUNK, _CHUNK)],
            ssems[slot],
        )
    for s in range(_NBUF):
        if store[s] is not None:
            store[s].wait()


def kernel(x, weight):
    idx = jnp.reshape(x, (_N,)).astype(jnp.int32)
    out = _emb_lookup(idx, weight)
    return jnp.reshape(out, (_B, _S, _D))
